# initial kernel scaffold (unmeasured)
import jax
import jax.numpy as jnp
from jax import lax
from jax.experimental import pallas as pl
from jax.experimental.pallas import tpu as pltpu

N_DEV = 4
M_PER = 1024
K_PER = 1024
N_TOT = 8192
N_BLK = 4096
N_STEPS_N = N_TOT // N_BLK


def kernel(x, w_mat):
    m_tot, k_per = x.shape
    k_tot, n_tot = w_mat.shape

    def body(x_ref, w_ref, out_ref, gather_ref, amax_ref, amax_gather,
             send_sems, recv_sems, amax_send_sems, amax_recv_sems):
        k = pl.program_id(0)
        n = pl.program_id(1)
        my = lax.axis_index("i")

        @pl.when(jnp.logical_and(k == 0, n == 0))
        def _():
            barrier_sem = pltpu.get_barrier_semaphore()
            for d in range(1, N_DEV):
                peer = lax.rem(my + d, N_DEV)
                pl.semaphore_signal(
                    barrier_sem, inc=1,
                    device_id=(peer,), device_id_type=pl.DeviceIdType.MESH,
                )
            pl.semaphore_wait(barrier_sem, N_DEV - 1)

            for d in range(1, N_DEV):
                peer = lax.rem(my + d, N_DEV)
                rdma = pltpu.make_async_remote_copy(
                    src_ref=x_ref.at[pl.ds(peer * M_PER, M_PER), :],
                    dst_ref=gather_ref.at[my],
                    send_sem=send_sems.at[d],
                    recv_sem=recv_sems.at[my],
                    device_id=(peer,),
                    device_id_type=pl.DeviceIdType.MESH,
                )
                rdma.start()

            gather_ref[my] = x_ref[pl.ds(my * M_PER, M_PER), :]

            out_ref[...] = jnp.zeros((M_PER, n_tot), jnp.float32)

        @pl.when(jnp.logical_and(k != my, n == 0))
        def _():
            recv = pltpu.make_async_remote_copy(
                src_ref=gather_ref.at[k],
                dst_ref=gather_ref.at[k],
                send_sem=send_sems.at[0],
                recv_sem=recv_sems.at[k],
                device_id=(my,),
                device_id_type=pl.DeviceIdType.MESH,
            )
            recv.wait_recv()

        lhs = gather_ref[k]
        rhs = w_ref[...]
        part = lax.dot_general(
            lhs, rhs, (((1,), (0,)), ((), ())),
            preferred_element_type=jnp.float32,
        )
        cur = out_ref[:, pl.ds(n * N_BLK, N_BLK)]
        out_ref[:, pl.ds(n * N_BLK, N_BLK)] = cur + part

        @pl.when(jnp.logical_and(k == N_DEV - 1, n == N_STEPS_N - 1))
        def _():
            for d in range(1, N_DEV):
                peer = lax.rem(my + d, N_DEV)
                rdma = pltpu.make_async_remote_copy(
                    src_ref=x_ref.at[pl.ds(peer * M_PER, M_PER), :],
                    dst_ref=gather_ref.at[my],
                    send_sem=send_sems.at[d],
                    recv_sem=recv_sems.at[my],
                    device_id=(peer,),
                    device_id_type=pl.DeviceIdType.MESH,
                )
                rdma.wait_send()

            y = jnp.maximum(out_ref[...], 0.0)
            local_amax = jnp.max(y)

            amax_ref[...] = jnp.full((8, 128), local_amax, jnp.float32)
            amax_gather[my] = amax_ref[...]
            for d in range(1, N_DEV):
                peer = lax.rem(my + d, N_DEV)
                rdma = pltpu.make_async_remote_copy(
                    src_ref=amax_ref,
                    dst_ref=amax_gather.at[my],
                    send_sem=amax_send_sems.at[d],
                    recv_sem=amax_recv_sems.at[my],
                    device_id=(peer,),
                    device_id_type=pl.DeviceIdType.MESH,
                )
                rdma.start()
            for d in range(1, N_DEV):
                src_pos = lax.rem(my + d, N_DEV)
                recv = pltpu.make_async_remote_copy(
                    src_ref=amax_ref,
                    dst_ref=amax_gather.at[src_pos],
                    send_sem=amax_send_sems.at[0],
                    recv_sem=amax_recv_sems.at[src_pos],
                    device_id=(my,),
                    device_id_type=pl.DeviceIdType.MESH,
                )
                recv.wait_recv()
            for d in range(1, N_DEV):
                rdma = pltpu.make_async_remote_copy(
                    src_ref=amax_ref,
                    dst_ref=amax_gather.at[my],
                    send_sem=amax_send_sems.at[d],
                    recv_sem=amax_recv_sems.at[my],
                    device_id=(my,),
                    device_id_type=pl.DeviceIdType.MESH,
                )
                rdma.wait_send()

            g_amax = jnp.max(amax_gather[...])
            scale = g_amax / 448.0
            q = (y * (448.0 / g_amax)).astype(jnp.float8_e4m3fn)
            out_ref[...] = q.astype(jnp.float32) * scale

    return pl.pallas_call(
        body,
        grid=(N_DEV, N_STEPS_N),
        out_shape=jax.ShapeDtypeStruct((M_PER, n_tot), jnp.float32),
        in_specs=[
            pl.BlockSpec((m_tot, k_per), lambda k, n: (0, 0)),
            pl.BlockSpec((K_PER, N_BLK), lambda k, n: (k, n)),
        ],
        out_specs=pl.BlockSpec((M_PER, n_tot), lambda k, n: (0, 0)),
        scratch_shapes=[
            pltpu.VMEM((N_DEV, M_PER, K_PER), jnp.bfloat16),
            pltpu.VMEM((8, 128), jnp.float32),
            pltpu.VMEM((N_DEV, 8, 128), jnp.float32),
            pltpu.SemaphoreType.DMA((N_DEV,)),
            pltpu.SemaphoreType.DMA((N_DEV,)),
            pltpu.SemaphoreType.DMA((N_DEV,)),
            pltpu.SemaphoreType.DMA((N_DEV,)),
        ],
        compiler_params=pltpu.CompilerParams(
            collective_id=0,
            dimension_semantics=("arbitrary", "arbitrary"),
        ),
    )(x, w_mat)


# baseline (device time: 170532 ns/iter reference)
import jax
import jax.numpy as jnp
from jax import lax
from jax.experimental import pallas as pl
from jax.experimental.pallas import tpu as pltpu

N_DEV = 4
M_PER = 1024
K_PER = 1024
N_TOT = 8192
N_BLK = 1024
NS = N_TOT // N_BLK
TOTAL = N_DEV * NS
N_BLK_B = 2048

SEND_OFFS = (3, 1, 2)


def _src_offset(s):
    return s + jnp.where(s == 2, 1, 0) - jnp.where(s == 3, 1, 0)


def _gemm_a2a(x, w_mat):
    def body(x_ref, w_ref, y_ref, amax_out, sendbuf, gather_ref, acc,
             wbuf, xtmp, amax_acc, xsem, wsems, ysems, send_sems,
             recv_sems):
        s = pl.program_id(0)
        n = pl.program_id(1)
        my = lax.axis_index("i")
        flat = s * NS + n
        slot = lax.rem(flat, 2)

        def w_copy(f, sl):
            s_i = lax.div(f, NS)
            n_i = lax.rem(f, NS)
            ksrc = lax.rem(my + _src_offset(s_i), N_DEV)
            return pltpu.make_async_copy(
                w_ref.at[pl.ds(ksrc * K_PER, K_PER),
                         pl.ds(n_i * N_BLK, N_BLK)],
                wbuf.at[sl],
                wsems.at[sl],
            )

        def x_send_rdma(idx, peer):
            return pltpu.make_async_remote_copy(
                src_ref=sendbuf.at[idx],
                dst_ref=gather_ref.at[my],
                send_sem=send_sems.at[idx],
                recv_sem=recv_sems.at[my],
                device_id=(peer,),
                device_id_type=pl.DeviceIdType.MESH,
            )

        @pl.when(flat == 0)
        def _():
            barrier_sem = pltpu.get_barrier_semaphore()
            for d in range(1, N_DEV):
                pl.semaphore_signal(
                    barrier_sem, inc=1,
                    device_id=(lax.rem(my + d, N_DEV),),
                    device_id_type=pl.DeviceIdType.MESH,
                )
            pl.semaphore_wait(barrier_sem, N_DEV - 1)

            w_copy(0, 0).start()

            cp = pltpu.make_async_copy(
                x_ref.at[pl.ds(my * M_PER, M_PER), :], xtmp, xsem)
            cp.start()
            cp.wait()
            gather_ref[my] = xtmp[...].astype(jnp.bfloat16)

            for idx, d in enumerate(SEND_OFFS):
                peer = lax.rem(my + d, N_DEV)
                cp = pltpu.make_async_copy(
                    x_ref.at[pl.ds(peer * M_PER, M_PER), :], xtmp, xsem)
                cp.start()
                cp.wait()
                sendbuf[idx] = xtmp[...].astype(jnp.bfloat16)
                x_send_rdma(idx, peer).start()

        @pl.when(flat < TOTAL - 1)
        def _():
            w_copy(flat + 1, lax.rem(flat + 1, 2)).start()

        ksrc = lax.rem(my + _src_offset(s), N_DEV)

        @pl.when(jnp.logical_and(s > 0, n == 0))
        def _():
            pltpu.make_async_remote_copy(
                src_ref=gather_ref.at[ksrc],
                dst_ref=gather_ref.at[ksrc],
                send_sem=send_sems.at[0],
                recv_sem=recv_sems.at[ksrc],
                device_id=(my,),
                device_id_type=pl.DeviceIdType.MESH,
            ).wait_recv()

        w_copy(flat, slot).wait()
        lhs = gather_ref[ksrc]
        rhs = wbuf[slot].astype(jnp.bfloat16)
        part = lax.dot_general(
            lhs, rhs, (((1,), (0,)), ((), ())),
            preferred_element_type=jnp.float32,
        )
        cols = pl.ds(n * N_BLK, N_BLK)

        @pl.when(s == 0)
        def _():
            acc[:, cols] = part

        @pl.when(jnp.logical_and(s > 0, s < N_DEV - 1))
        def _():
            acc[:, cols] = acc[:, cols] + part

        @pl.when(s == N_DEV - 1)
        def _():
            yb = jnp.maximum(acc[:, cols] + part, 0.0)
            acc[:, cols] = yb
            m = jnp.max(yb)

            @pl.when(n == 0)
            def _():
                amax_acc[...] = jnp.full((8, 128), m, jnp.float32)

            @pl.when(n > 0)
            def _():
                amax_acc[...] = jnp.maximum(amax_acc[...], m)

            ysl = lax.rem(n, 2)

            @pl.when(n >= 2)
            def _():
                pltpu.make_async_copy(
                    acc.at[:, pl.ds((n - 2) * N_BLK, N_BLK)],
                    y_ref.at[:, pl.ds((n - 2) * N_BLK, N_BLK)],
                    ysems.at[ysl],
                ).wait()

            pltpu.make_async_copy(
                acc.at[:, cols], y_ref.at[:, cols], ysems.at[ysl]
            ).start()

            @pl.when(n == NS - 1)
            def _():
                for j in range(2):
                    pltpu.make_async_copy(
                        acc.at[:, pl.ds((NS - 2 + j) * N_BLK, N_BLK)],
                        y_ref.at[:, pl.ds((NS - 2 + j) * N_BLK, N_BLK)],
                        ysems.at[j],
                    ).wait()
                for idx, d in enumerate(SEND_OFFS):
                    x_send_rdma(idx, lax.rem(my + d, N_DEV)).wait_send()
                amax_out[...] = amax_acc[...]

    return pl.pallas_call(
        body,
        grid=(N_DEV, NS),
        out_shape=[
            jax.ShapeDtypeStruct((M_PER, N_TOT), jnp.float32),
            jax.ShapeDtypeStruct((8, 128), jnp.float32),
        ],
        in_specs=[
            pl.BlockSpec(memory_space=pl.ANY),
            pl.BlockSpec(memory_space=pl.ANY),
        ],
        out_specs=[
            pl.BlockSpec(memory_space=pl.ANY),
            pl.BlockSpec((8, 128), lambda s, n: (0, 0)),
        ],
        scratch_shapes=[
            pltpu.VMEM((3, M_PER, K_PER), jnp.bfloat16),
            pltpu.VMEM((N_DEV, M_PER, K_PER), jnp.bfloat16),
            pltpu.VMEM((M_PER, N_TOT), jnp.float32),
            pltpu.VMEM((2, K_PER, N_BLK), jnp.float32),
            pltpu.VMEM((M_PER, K_PER), jnp.float32),
            pltpu.VMEM((8, 128), jnp.float32),
            pltpu.SemaphoreType.DMA,
            pltpu.SemaphoreType.DMA((2,)),
            pltpu.SemaphoreType.DMA((2,)),
            pltpu.SemaphoreType.DMA((3,)),
            pltpu.SemaphoreType.DMA((N_DEV,)),
        ],
        compiler_params=pltpu.CompilerParams(
            collective_id=0,
            dimension_semantics=("arbitrary", "arbitrary"),
            vmem_limit_bytes=100 * 1024 * 1024,
        ),
    )(x, w_mat)


def _quant_epilogue(y, amax_tile):
    nsb = N_TOT // N_BLK_B

    def body(y_ref, amax_ref, out_ref, agather, gscr, asend, arecv):
        n = pl.program_id(0)
        my = lax.axis_index("i")

        @pl.when(n == 0)
        def _():
            barrier_sem = pltpu.get_barrier_semaphore()
            for d in range(1, N_DEV):
                pl.semaphore_signal(
                    barrier_sem, inc=1,
                    device_id=(lax.rem(my + d, N_DEV),),
                    device_id_type=pl.DeviceIdType.MESH,
                )
            pl.semaphore_wait(barrier_sem, N_DEV - 1)

            agather[my] = amax_ref[...]
            for d in range(1, N_DEV):
                pltpu.make_async_remote_copy(
                    src_ref=amax_ref,
                    dst_ref=agather.at[my],
                    send_sem=asend.at[d - 1],
                    recv_sem=arecv.at[my],
                    device_id=(lax.rem(my + d, N_DEV),),
                    device_id_type=pl.DeviceIdType.MESH,
                ).start()
            for d in range(1, N_DEV):
                src_pos = lax.rem(my + d, N_DEV)
                pltpu.make_async_remote_copy(
                    src_ref=amax_ref,
                    dst_ref=agather.at[src_pos],
                    send_sem=asend.at[0],
                    recv_sem=arecv.at[src_pos],
                    device_id=(my,),
                    device_id_type=pl.DeviceIdType.MESH,
                ).wait_recv()
            for d in range(1, N_DEV):
                pltpu.make_async_remote_copy(
                    src_ref=amax_ref,
                    dst_ref=agather.at[my],
                    send_sem=asend.at[d - 1],
                    recv_sem=arecv.at[my],
                    device_id=(my,),
                    device_id_type=pl.DeviceIdType.MESH,
                ).wait_send()
            gscr[...] = jnp.full(
                (8, 128), jnp.max(agather[...]), jnp.float32)

        g = gscr[0, 0]
        yb = y_ref[...]
        q = (yb * (448.0 / g)).astype(jnp.float8_e4m3fn)
        out_ref[...] = q.astype(jnp.float32) * (g / 448.0)

    return pl.pallas_call(
        body,
        grid=(nsb,),
        out_shape=jax.ShapeDtypeStruct((M_PER, N_TOT), jnp.float32),
        in_specs=[
            pl.BlockSpec((M_PER, N_BLK_B), lambda n: (0, n)),
            pl.BlockSpec((8, 128), lambda n: (0, 0)),
        ],
        out_specs=pl.BlockSpec((M_PER, N_BLK_B), lambda n: (0, n)),
        scratch_shapes=[
            pltpu.VMEM((N_DEV, 8, 128), jnp.float32),
            pltpu.VMEM((8, 128), jnp.float32),
            pltpu.SemaphoreType.DMA((3,)),
            pltpu.SemaphoreType.DMA((N_DEV,)),
        ],
        compiler_params=pltpu.CompilerParams(
            collective_id=1,
            dimension_semantics=("arbitrary",),
            vmem_limit_bytes=100 * 1024 * 1024,
        ),
    )(y, amax_tile)


def kernel(x, w_mat):
    y, amax_tile = _gemm_a2a(x, w_mat)
    return _quant_epilogue(y, amax_tile)


# device time: 161688 ns/iter; 1.0547x vs baseline; 1.0547x over previous
import jax
import jax.numpy as jnp
from jax import lax
from jax.experimental import pallas as pl
from jax.experimental.pallas import tpu as pltpu

N_DEV = 4
M_PER = 1024
K_PER = 1024
N_TOT = 8192
N_BLK = 512
NS = N_TOT // N_BLK
TOTAL = N_DEV * NS
W_SLOTS = 4

DIRECT_OFFS = (3, 1)
DIAG_OFF = 2


def _src_offset(s):
    return s + jnp.where(s == 2, 1, 0) - jnp.where(s == 3, 1, 0)


def kernel(x, w_mat):
    def body(x_ref, w_ref, out_ref, sendbuf, gather_ref, acc, wbuf, xtmp,
             agather, xsem, wsems, ysems, send_sems, recv_sems, asend,
             arecv):
        s = pl.program_id(0)
        n = pl.program_id(1)
        my = lax.axis_index("i")
        flat = s * NS + n

        def w_copy(f, sl):
            s_i = lax.div(f, NS)
            n_i = lax.rem(f, NS)
            ksrc = lax.rem(my + _src_offset(s_i), N_DEV)
            return pltpu.make_async_copy(
                w_ref.at[pl.ds(ksrc * K_PER, K_PER),
                         pl.ds(n_i * N_BLK, N_BLK)],
                wbuf.at[sl],
                wsems.at[sl],
            )

        def x_send_rdma(idx, peer):
            return pltpu.make_async_remote_copy(
                src_ref=sendbuf.at[idx],
                dst_ref=gather_ref.at[my],
                send_sem=send_sems.at[idx],
                recv_sem=recv_sems.at[my],
                device_id=(peer,),
                device_id_type=pl.DeviceIdType.MESH,
            )

        def stage_and_send(idx, d):
            peer = lax.rem(my + d, N_DEV)
            cp = pltpu.make_async_copy(
                x_ref.at[pl.ds(peer * M_PER, M_PER), :], xtmp, xsem)
            cp.start()
            cp.wait()
            sendbuf[idx] = xtmp[...].astype(jnp.bfloat16)
            x_send_rdma(idx, peer).start()

        @pl.when(flat == 0)
        def _():
            barrier_sem = pltpu.get_barrier_semaphore()
            for d in range(1, N_DEV):
                pl.semaphore_signal(
                    barrier_sem, inc=1,
                    device_id=(lax.rem(my + d, N_DEV),),
                    device_id_type=pl.DeviceIdType.MESH,
                )
            pl.semaphore_wait(barrier_sem, N_DEV - 1)

            for sl in range(W_SLOTS - 1):
                w_copy(sl, sl).start()

            cp = pltpu.make_async_copy(
                x_ref.at[pl.ds(my * M_PER, M_PER), :], xtmp, xsem)
            cp.start()
            cp.wait()
            gather_ref[my] = xtmp[...].astype(jnp.bfloat16)

            for idx, d in enumerate(DIRECT_OFFS):
                stage_and_send(idx, d)

        @pl.when(jnp.logical_and(s == 1, n == 0))
        def _():
            for idx, d in enumerate(DIRECT_OFFS):
                x_send_rdma(idx, lax.rem(my + d, N_DEV)).wait_send()
            stage_and_send(2, DIAG_OFF)

        @pl.when(flat < TOTAL - (W_SLOTS - 1))
        def _():
            f = flat + W_SLOTS - 1
            w_copy(f, lax.rem(f, W_SLOTS)).start()

        ksrc = lax.rem(my + _src_offset(s), N_DEV)

        @pl.when(jnp.logical_and(s > 0, n == 0))
        def _():
            pltpu.make_async_remote_copy(
                src_ref=gather_ref.at[ksrc],
                dst_ref=gather_ref.at[ksrc],
                send_sem=send_sems.at[0],
                recv_sem=recv_sems.at[ksrc],
                device_id=(my,),
                device_id_type=pl.DeviceIdType.MESH,
            ).wait_recv()

        slot = lax.rem(flat, W_SLOTS)
        w_copy(flat, slot).wait()
        lhs = gather_ref[ksrc]
        rhs = wbuf[slot].astype(jnp.bfloat16)
        part = lax.dot_general(
            lhs, rhs, (((1,), (0,)), ((), ())),
            preferred_element_type=jnp.float32,
        )
        cols = pl.ds(n * N_BLK, N_BLK)

        @pl.when(s == 0)
        def _():
            acc[:, cols] = part

        @pl.when(jnp.logical_and(s > 0, s < N_DEV - 1))
        def _():
            acc[:, cols] = acc[:, cols] + part

        @pl.when(s == N_DEV - 1)
        def _():
            yb = jnp.maximum(acc[:, cols] + part, 0.0)
            acc[:, cols] = yb
            m = jnp.max(yb)

            @pl.when(n == 0)
            def _():
                agather[my] = jnp.full((8, 128), m, jnp.float32)

            @pl.when(n > 0)
            def _():
                agather[my] = jnp.maximum(
                    agather[my], jnp.full((8, 128), m, jnp.float32))

        @pl.when(flat == TOTAL - 1)
        def _():
            x_send_rdma(2, lax.rem(my + DIAG_OFF, N_DEV)).wait_send()

            def amax_rdma(sem_i, peer):
                return pltpu.make_async_remote_copy(
                    src_ref=agather.at[my],
                    dst_ref=agather.at[my],
                    send_sem=asend.at[sem_i],
                    recv_sem=arecv.at[my],
                    device_id=(peer,),
                    device_id_type=pl.DeviceIdType.MESH,
                )

            for d in range(1, N_DEV):
                amax_rdma(d - 1, lax.rem(my + d, N_DEV)).start()
            for d in range(1, N_DEV):
                src_pos = lax.rem(my + d, N_DEV)
                pltpu.make_async_remote_copy(
                    src_ref=agather.at[src_pos],
                    dst_ref=agather.at[src_pos],
                    send_sem=asend.at[0],
                    recv_sem=arecv.at[src_pos],
                    device_id=(my,),
                    device_id_type=pl.DeviceIdType.MESH,
                ).wait_recv()
            for d in range(1, N_DEV):
                amax_rdma(d - 1, lax.rem(my + d, N_DEV)).wait_send()

            g = jnp.max(agather[...])
            inv = 448.0 / g
            sc = g / 448.0

            for j in range(NS):
                jcols = pl.ds(j * N_BLK, N_BLK)
                q = (acc[:, jcols] * inv).astype(jnp.float8_e4m3fn)
                if j >= 2:
                    pltpu.make_async_copy(
                        acc.at[:, pl.ds((j - 2) * N_BLK, N_BLK)],
                        out_ref.at[:, pl.ds((j - 2) * N_BLK, N_BLK)],
                        ysems.at[j % 2],
                    ).wait()
                acc[:, jcols] = q.astype(jnp.float32) * sc
                pltpu.make_async_copy(
                    acc.at[:, jcols], out_ref.at[:, jcols],
                    ysems.at[j % 2],
                ).start()
            for j in range(2):
                pltpu.make_async_copy(
                    acc.at[:, pl.ds((NS - 2 + j) * N_BLK, N_BLK)],
                    out_ref.at[:, pl.ds((NS - 2 + j) * N_BLK, N_BLK)],
                    ysems.at[j],
                ).wait()

    return pl.pallas_call(
        body,
        grid=(N_DEV, NS),
        out_shape=jax.ShapeDtypeStruct((M_PER, N_TOT), jnp.float32),
        in_specs=[
            pl.BlockSpec(memory_space=pl.ANY),
            pl.BlockSpec(memory_space=pl.ANY),
        ],
        out_specs=pl.BlockSpec(memory_space=pl.ANY),
        scratch_shapes=[
            pltpu.VMEM((3, M_PER, K_PER), jnp.bfloat16),
            pltpu.VMEM((N_DEV, M_PER, K_PER), jnp.bfloat16),
            pltpu.VMEM((M_PER, N_TOT), jnp.float32),
            pltpu.VMEM((W_SLOTS, K_PER, N_BLK), jnp.float32),
            pltpu.VMEM((M_PER, K_PER), jnp.float32),
            pltpu.VMEM((N_DEV, 8, 128), jnp.float32),
            pltpu.SemaphoreType.DMA,
            pltpu.SemaphoreType.DMA((W_SLOTS,)),
            pltpu.SemaphoreType.DMA((2,)),
            pltpu.SemaphoreType.DMA((3,)),
            pltpu.SemaphoreType.DMA((N_DEV,)),
            pltpu.SemaphoreType.DMA((3,)),
            pltpu.SemaphoreType.DMA((N_DEV,)),
        ],
        compiler_params=pltpu.CompilerParams(
            collective_id=0,
            dimension_semantics=("arbitrary", "arbitrary"),
            vmem_limit_bytes=100 * 1024 * 1024,
        ),
    )(x, w_mat)


# device time: 146072 ns/iter; 1.1675x vs baseline; 1.1069x over previous
import jax
import jax.numpy as jnp
from jax import lax
from jax.experimental import pallas as pl
from jax.experimental.pallas import tpu as pltpu

N_DEV = 4
M_PER = 1024
K_PER = 1024
N_TOT = 8192
N_BLK = 1024
NS = N_TOT // N_BLK
TOTAL = N_DEV * NS
Q_BLK = 1024
QS = N_TOT // Q_BLK

SEND_OFFS = (3, 1, 2)


def _src_offset(s):
    return s + jnp.where(s == 2, 1, 0) - jnp.where(s == 3, 1, 0)


def kernel(x, w_mat):
    def body(x_ref, w_ref, out_ref, sendbuf, gather_ref, acc, wbuf,
             agather, xsems, wsems, ysems, send_sems, recv_sems, asend,
             arecv):
        s = pl.program_id(0)
        n = pl.program_id(1)
        my = lax.axis_index("i")
        flat = s * NS + n

        def w_copy(f, sl):
            s_i = lax.div(f, NS)
            n_i = lax.rem(f, NS)
            ksrc = lax.rem(my + _src_offset(s_i), N_DEV)
            return pltpu.make_async_copy(
                w_ref.at[pl.ds(ksrc * K_PER, K_PER),
                         pl.ds(n_i * N_BLK, N_BLK)],
                wbuf.at[sl],
                wsems.at[sl],
            )

        def x_send_rdma(idx, peer):
            return pltpu.make_async_remote_copy(
                src_ref=sendbuf.at[idx],
                dst_ref=gather_ref.at[my],
                send_sem=send_sems.at[idx],
                recv_sem=recv_sems.at[my],
                device_id=(peer,),
                device_id_type=pl.DeviceIdType.MESH,
            )

        def x_block_copy(row_blk, sl):
            return pltpu.make_async_copy(
                x_ref.at[pl.ds(row_blk * M_PER, M_PER), :],
                wbuf.at[sl],
                xsems.at[sl],
            )

        @pl.when(flat == 0)
        def _():
            barrier_sem = pltpu.get_barrier_semaphore()
            for d in range(1, N_DEV):
                pl.semaphore_signal(
                    barrier_sem, inc=1,
                    device_id=(lax.rem(my + d, N_DEV),),
                    device_id_type=pl.DeviceIdType.MESH,
                )
            pl.semaphore_wait(barrier_sem, N_DEV - 1)

            own = x_block_copy(my, 0)
            own.start()
            left = lax.rem(my + SEND_OFFS[0], N_DEV)
            cpl = x_block_copy(left, 1)
            cpl.start()
            own.wait()
            gather_ref[my] = wbuf[0].astype(jnp.bfloat16)
            right = lax.rem(my + SEND_OFFS[1], N_DEV)
            cpr = x_block_copy(right, 0)
            cpr.start()
            cpl.wait()
            sendbuf[0] = wbuf[1].astype(jnp.bfloat16)
            x_send_rdma(0, left).start()
            diag = lax.rem(my + SEND_OFFS[2], N_DEV)
            cpd = x_block_copy(diag, 1)
            cpd.start()
            cpr.wait()
            sendbuf[1] = wbuf[0].astype(jnp.bfloat16)
            x_send_rdma(1, right).start()
            cpd.wait()
            sendbuf[2] = wbuf[1].astype(jnp.bfloat16)

            w_copy(0, 0).start()
            w_copy(1, 1).start()

        @pl.when(jnp.logical_and(s == 1, n == 0))
        def _():
            for idx in range(2):
                x_send_rdma(
                    idx, lax.rem(my + SEND_OFFS[idx], N_DEV)).wait_send()
            x_send_rdma(2, lax.rem(my + SEND_OFFS[2], N_DEV)).start()

        @pl.when(jnp.logical_and(flat > 0, flat < TOTAL - 1))
        def _():
            f = flat + 1
            w_copy(f, lax.rem(f, 2)).start()

        ksrc = lax.rem(my + _src_offset(s), N_DEV)

        @pl.when(jnp.logical_and(s > 0, n == 0))
        def _():
            pltpu.make_async_remote_copy(
                src_ref=gather_ref.at[ksrc],
                dst_ref=gather_ref.at[ksrc],
                send_sem=send_sems.at[0],
                recv_sem=recv_sems.at[ksrc],
                device_id=(my,),
                device_id_type=pl.DeviceIdType.MESH,
            ).wait_recv()

        slot = lax.rem(flat, 2)
        w_copy(flat, slot).wait()
        lhs = gather_ref[ksrc]
        rhs = wbuf[slot].astype(jnp.bfloat16)
        part = lax.dot_general(
            lhs, rhs, (((1,), (0,)), ((), ())),
            preferred_element_type=jnp.float32,
        )
        cols = pl.ds(n * N_BLK, N_BLK)

        @pl.when(s == 0)
        def _():
            acc[:, cols] = part

        @pl.when(jnp.logical_and(s > 0, s < N_DEV - 1))
        def _():
            acc[:, cols] = acc[:, cols] + part

        @pl.when(s == N_DEV - 1)
        def _():
            yb = jnp.maximum(acc[:, cols] + part, 0.0)
            acc[:, cols] = yb
            m = jnp.full((8, 128), jnp.max(yb), jnp.float32)

            @pl.when(n == 0)
            def _():
                agather[my] = m

            @pl.when(n > 0)
            def _():
                agather[my] = jnp.maximum(agather[my], m)

        @pl.when(flat == TOTAL - 1)
        def _():
            x_send_rdma(2, lax.rem(my + SEND_OFFS[2], N_DEV)).wait_send()

            def amax_rdma(sem_i, peer):
                return pltpu.make_async_remote_copy(
                    src_ref=agather.at[my],
                    dst_ref=agather.at[my],
                    send_sem=asend.at[sem_i],
                    recv_sem=arecv.at[my],
                    device_id=(peer,),
                    device_id_type=pl.DeviceIdType.MESH,
                )

            for d in range(1, N_DEV):
                amax_rdma(d - 1, lax.rem(my + d, N_DEV)).start()
            for d in range(1, N_DEV):
                src_pos = lax.rem(my + d, N_DEV)
                pltpu.make_async_remote_copy(
                    src_ref=agather.at[src_pos],
                    dst_ref=agather.at[src_pos],
                    send_sem=asend.at[0],
                    recv_sem=arecv.at[src_pos],
                    device_id=(my,),
                    device_id_type=pl.DeviceIdType.MESH,
                ).wait_recv()
            for d in range(1, N_DEV):
                amax_rdma(d - 1, lax.rem(my + d, N_DEV)).wait_send()

            g = jnp.max(agather[...])
            inv = 448.0 / g
            sc = g / 448.0

            for j in range(QS):
                jcols = pl.ds(j * Q_BLK, Q_BLK)
                q = (acc[:, jcols] * inv).astype(jnp.float8_e4m3fn)
                if j >= 2:
                    pltpu.make_async_copy(
                        acc.at[:, pl.ds((j - 2) * Q_BLK, Q_BLK)],
                        out_ref.at[:, pl.ds((j - 2) * Q_BLK, Q_BLK)],
                        ysems.at[j % 2],
                    ).wait()
                acc[:, jcols] = q.astype(jnp.float32) * sc
                pltpu.make_async_copy(
                    acc.at[:, jcols], out_ref.at[:, jcols],
                    ysems.at[j % 2],
                ).start()
            for j in range(2):
                pltpu.make_async_copy(
                    acc.at[:, pl.ds((QS - 2 + j) * Q_BLK, Q_BLK)],
                    out_ref.at[:, pl.ds((QS - 2 + j) * Q_BLK, Q_BLK)],
                    ysems.at[j],
                ).wait()

    return pl.pallas_call(
        body,
        grid=(N_DEV, NS),
        out_shape=jax.ShapeDtypeStruct((M_PER, N_TOT), jnp.float32),
        in_specs=[
            pl.BlockSpec(memory_space=pl.ANY),
            pl.BlockSpec(memory_space=pl.ANY),
        ],
        out_specs=pl.BlockSpec(memory_space=pltpu.MemorySpace.HBM),
        scratch_shapes=[
            pltpu.VMEM((3, M_PER, K_PER), jnp.bfloat16),
            pltpu.VMEM((N_DEV, M_PER, K_PER), jnp.bfloat16),
            pltpu.VMEM((M_PER, N_TOT), jnp.float32),
            pltpu.VMEM((2, K_PER, N_BLK), jnp.float32),
            pltpu.VMEM((N_DEV, 8, 128), jnp.float32),
            pltpu.SemaphoreType.DMA((2,)),
            pltpu.SemaphoreType.DMA((2,)),
            pltpu.SemaphoreType.DMA((2,)),
            pltpu.SemaphoreType.DMA((3,)),
            pltpu.SemaphoreType.DMA((N_DEV,)),
            pltpu.SemaphoreType.DMA((3,)),
            pltpu.SemaphoreType.DMA((N_DEV,)),
        ],
        compiler_params=pltpu.CompilerParams(
            collective_id=0,
            dimension_semantics=("arbitrary", "arbitrary"),
            vmem_limit_bytes=100 * 1024 * 1024,
        ),
    )(x, w_mat)
